# Initial kernel scaffold; baseline (speedup 1.0000x reference)
import jax
import jax.numpy as jnp
from jax.experimental import pallas as pl


def kernel(x, edge_index_0, edge_weight_0, edge_index_1, edge_weight_1, edge_index_2, edge_weight_2, W_aff, b_aff, ln_gamma, ln_beta):
    raise NotImplementedError("write your pallas kernel here")


# speedup vs baseline: 3.5488x; 3.5488x over previous
import functools

import jax
import jax.numpy as jnp
from jax import lax
from jax.experimental import pallas as pl
from jax.experimental.pallas import tpu as pltpu
from jax.experimental.pallas import tpu_sc as plsc

N_NODES = 10000
D = 128
N_EDGES = 320000

NC = 2
NS = 16
NW = NC * NS
EPT = N_EDGES // NW
EPB = 80
NBLK = EPT // EPB
N_PAD = 10240
RPS = N_PAD // NS
ZCH = 128


def _scale_rows(rows, wv, scale, n_groups):
    def grp(g, _):
        w16 = wv[pl.ds(g * 16, 16)] * scale
        for e in range(16):
            wb = w16[e]
            r = g * 16 + e
            for j in range(8):
                sl = pl.ds(16 * j, 16)
                rows[r, sl] = rows[r, sl] * wb
        return 0

    lax.fori_loop(0, n_groups, grp, 0, unroll=False)


def _edge_pass(src, dst, w, tbl_hbm, acc, idx_s, idx_d, wv, rows, sem, tile,
               scale):
    def blk(b, _):
        base = tile * EPT + b * EPB
        pltpu.sync_copy(src.at[pl.ds(base, EPB)], idx_s)
        pltpu.sync_copy(dst.at[pl.ds(base, EPB)], idx_d)
        pltpu.sync_copy(w.at[pl.ds(base, EPB)], wv)
        pltpu.async_copy(tbl_hbm.at[idx_s], rows, sem).wait()
        _scale_rows(rows, wv, scale, EPB // 16)
        pltpu.sync_copy(rows, acc.at[idx_d], add=True)
        return 0

    lax.fori_loop(0, NBLK, blk, 0, unroll=False)


def _zero_acc(acc, zb, s):
    zeros = jnp.zeros((16,), jnp.float32)

    def zrow(i, _):
        for j in range(8):
            zb[i, pl.ds(16 * j, 16)] = zeros
        return 0

    lax.fori_loop(0, ZCH, zrow, 0, unroll=False)
    for k in range(RPS // ZCH):
        pltpu.sync_copy(zb, acc.at[pl.ds(s * RPS + k * ZCH, ZCH)])


def _copy_out(acc, out_hbm, c, s):
    for k in range(RPS // ZCH):
        r0 = s * RPS + k * ZCH
        pltpu.sync_copy(acc.at[pl.ds(r0, ZCH)], out_hbm.at[c, pl.ds(r0, ZCH)])


_SC_MESH = plsc.VectorSubcoreMesh(core_axis_name="c", subcore_axis_name="s")

_SPMM_SCRATCH = [
    pltpu.VMEM((EPB,), jnp.int32),
    pltpu.VMEM((EPB,), jnp.int32),
    pltpu.VMEM((EPB,), jnp.float32),
    pltpu.VMEM((EPB, D), jnp.float32),
    pltpu.VMEM((ZCH, D), jnp.float32),
    pltpu.VMEM_SHARED((N_PAD, D), jnp.float32),
    pltpu.SemaphoreType.DMA,
]


@functools.partial(
    pl.kernel,
    out_type=jax.ShapeDtypeStruct((NC, N_PAD, D), jnp.float32),
    mesh=_SC_MESH,
    scratch_types=_SPMM_SCRATCH,
)
def _sc_spmm_pair(src0, dst0, w0, src1, dst1, w1, h_hbm, out_hbm,
                  idx_s, idx_d, wv, rows, zb, acc, sem):
    c = lax.axis_index("c")
    s = lax.axis_index("s")
    tile = c * NS + s
    _zero_acc(acc, zb, s)
    plsc.subcore_barrier()
    _edge_pass(src0, dst0, w0, h_hbm, acc, idx_s, idx_d, wv, rows, sem, tile,
               0.5)
    _edge_pass(src1, dst1, w1, h_hbm, acc, idx_s, idx_d, wv, rows, sem, tile,
               0.5)
    plsc.subcore_barrier()
    _copy_out(acc, out_hbm, c, s)


@functools.partial(
    pl.kernel,
    out_type=jax.ShapeDtypeStruct((NC, N_PAD, D), jnp.float32),
    mesh=_SC_MESH,
    scratch_types=_SPMM_SCRATCH,
)
def _sc_spmm_single(src2, dst2, w2, s1_hbm, out_hbm,
                    idx_s, idx_d, wv, rows, zb, acc, sem):
    c = lax.axis_index("c")
    s = lax.axis_index("s")
    tile = c * NS + s
    _zero_acc(acc, zb, s)
    plsc.subcore_barrier()
    _edge_pass(src2, dst2, w2, s1_hbm, acc, idx_s, idx_d, wv, rows, sem, tile,
               1.0)
    plsc.subcore_barrier()
    _copy_out(acc, out_hbm, c, s)


_ROWS_BLK = 1000


def _tc_affine_body(x_ref, w_ref, b_ref, o_ref):
    o_ref[...] = lax.dot_general(
        x_ref[...], w_ref[...],
        (((1,), (1,)), ((), ())),
        preferred_element_type=jnp.float32,
    ) + b_ref[...]


def _tc_affine(x, W, b):
    return pl.pallas_call(
        _tc_affine_body,
        out_shape=jax.ShapeDtypeStruct((N_NODES, D), jnp.float32),
        grid=(N_NODES // _ROWS_BLK,),
        in_specs=[
            pl.BlockSpec((_ROWS_BLK, D), lambda i: (i, 0)),
            pl.BlockSpec((D, D), lambda i: (0, 0)),
            pl.BlockSpec((1, D), lambda i: (0, 0)),
        ],
        out_specs=pl.BlockSpec((_ROWS_BLK, D), lambda i: (i, 0)),
    )(x, W, b.reshape(1, D))


def _tc_sum_pair_body(p_ref, o_ref):
    o_ref[...] = p_ref[0] + p_ref[1]


def _tc_sum_pair(p):
    return pl.pallas_call(
        _tc_sum_pair_body,
        out_shape=jax.ShapeDtypeStruct((N_NODES, D), jnp.float32),
        grid=(N_NODES // _ROWS_BLK,),
        in_specs=[pl.BlockSpec((NC, _ROWS_BLK, D), lambda i: (0, i, 0))],
        out_specs=pl.BlockSpec((_ROWS_BLK, D), lambda i: (i, 0)),
    )(p)


def _tc_finish_body(p_ref, h_ref, g_ref, bt_ref, o_ref):
    t = p_ref[0] + p_ref[1] + h_ref[...]
    mu = jnp.mean(t, axis=-1, keepdims=True)
    var = jnp.mean((t - mu) ** 2, axis=-1, keepdims=True)
    t = (t - mu) * lax.rsqrt(var + 1e-5) * g_ref[...] + bt_ref[...]
    o_ref[...] = t * 0.5 * (1.0 + lax.erf(t * (2.0 ** -0.5)))


def _tc_finish(p, h, gamma, beta):
    return pl.pallas_call(
        _tc_finish_body,
        out_shape=jax.ShapeDtypeStruct((N_NODES, D), jnp.float32),
        grid=(N_NODES // _ROWS_BLK,),
        in_specs=[
            pl.BlockSpec((NC, _ROWS_BLK, D), lambda i: (0, i, 0)),
            pl.BlockSpec((_ROWS_BLK, D), lambda i: (i, 0)),
            pl.BlockSpec((1, D), lambda i: (0, 0)),
            pl.BlockSpec((1, D), lambda i: (0, 0)),
        ],
        out_specs=pl.BlockSpec((_ROWS_BLK, D), lambda i: (i, 0)),
    )(p, h, gamma.reshape(1, D), beta.reshape(1, D))


def kernel(x, edge_index_0, edge_weight_0, edge_index_1, edge_weight_1,
           edge_index_2, edge_weight_2, W_aff, b_aff, ln_gamma, ln_beta):
    s0 = edge_index_0[0].astype(jnp.int32)
    d0 = edge_index_0[1].astype(jnp.int32)
    s1i = edge_index_1[0].astype(jnp.int32)
    d1 = edge_index_1[1].astype(jnp.int32)
    s2 = edge_index_2[0].astype(jnp.int32)
    d2 = edge_index_2[1].astype(jnp.int32)

    h = _tc_affine(x, W_aff, b_aff)
    p01 = _sc_spmm_pair(s0, d0, edge_weight_0, s1i, d1, edge_weight_1, h)
    s1 = _tc_sum_pair(p01)
    p2 = _sc_spmm_single(s2, d2, edge_weight_2, s1)
    return _tc_finish(p2, h, ln_gamma, ln_beta)
